# baseline (speedup 1.0000x reference)
import math

import jax
import jax.numpy as jnp
from jax.experimental import pallas as pl

C = 96
H = W = 384
WS = 8
HEADS = 4
DH = C // HEADS
NWH = H // WS
NWW = W // WS
NW = NWH * NWW
K = NW // 2
EPS = 1e-6
SCALE = 1.0 / math.sqrt(float(DH))
WPB = 16
NCG = NWW // WPB


def _ln_cf3(xt, w_ref, b_ref):
    u = jnp.mean(xt, axis=0, keepdims=True)
    s = jnp.mean((xt - u) * (xt - u), axis=0, keepdims=True)
    xn = (xt - u) * jax.lax.rsqrt(s + EPS)
    return w_ref[:][:, :, None] * xn + b_ref[:][:, :, None]


def _mm3(w, x3):
    return jax.lax.dot_general(w, x3, (((1,), (0,)), ((), ())),
                               preferred_element_type=jnp.float32)


def _pred_kernel(x_ref, lnw_ref, lnb_ref, piw_ref, pib_ref, m1_ref, b1_ref,
                 m2d_ref, d_ref):
    xt = x_ref[:, 0, :, :]
    xn = _ln_cf3(xt, lnw_ref, lnb_ref)
    f = _mm3(piw_ref[:], xn) + pib_ref[:][:, :, None]
    f = jax.nn.leaky_relu(f, 0.1)
    fm = jnp.mean(f, axis=0)
    fm3 = fm.reshape(8, NWW, 8)
    h1 = jnp.zeros((8, NWW), jnp.float32) + b1_ref[:]
    for i in range(8):
        h1 = h1 + jax.lax.dot_general(m1_ref[i], fm3[i],
                                      (((0,), (1,)), ((), ())),
                                      preferred_element_type=jnp.float32)
    h1 = jax.nn.leaky_relu(h1, 0.1)
    d = jax.lax.dot_general(m2d_ref[:], h1, (((1,), (0,)), ((), ())),
                            preferred_element_type=jnp.float32)
    d_ref[0, :, :] = d


def _mask_kernel(d_ref, mask_ref):
    r = pl.program_id(0)
    dall = d_ref[:]
    di = d_ref[pl.ds(r, 1), :]
    dj3 = dall[:, :, None]
    di3 = di[None, :, :]
    jidx = (jax.lax.broadcasted_iota(jnp.int32, (18, 128, 128), 0) * 128
            + jax.lax.broadcasted_iota(jnp.int32, (18, 128, 128), 1))
    iidx = (jax.lax.broadcasted_iota(jnp.int32, (18, 128, 128), 2)
            + r * 128)
    ahead = (dj3 > di3) | ((dj3 == di3) & (jidx < iidx))
    cnt = jnp.sum(ahead.astype(jnp.float32), axis=(0, 1))
    mask_ref[0, 0, :] = (cnt < float(K)).astype(jnp.float32)


def _block_kernel(x_ref, mask_ref, lnw_ref, lnb_ref, wv_ref, bv_ref,
                  piw_ref, pib_ref, psw_ref, psb_ref, wqt_ref, wkt_ref,
                  wo_ref, bo_ref, ln2w_ref, ln2b_ref, w1t_ref, b1_ref,
                  w2t_ref, b2_ref, out_ref):
    r = pl.program_id(0)
    cg = pl.program_id(1)
    xt = x_ref[:, 0, :, :]
    xn = _ln_cf3(xt, lnw_ref, lnb_ref)
    v = _mm3(wv_ref[:], xn) + bv_ref[:][:, :, None]
    f = jax.nn.leaky_relu(_mm3(piw_ref[:], xn) + pib_ref[:][:, :, None], 0.1)
    sa = jax.nn.sigmoid(_mm3(psw_ref[:], f) + psb_ref[0, 0])
    vsa = v * sa
    q = _mm3(wqt_ref[:], xn)
    k = _mm3(wkt_ref[:], xn)
    mstrip = mask_ref[r, pl.ds(cg, 1), :]
    parts = []
    for wloc in range(WPB):
        sl = slice(wloc * 8, wloc * 8 + 8)
        qw = q[:, :, sl].reshape(C, 64)
        kw = k[:, :, sl].reshape(C, 64)
        vw = vsa[:, :, sl].reshape(C, 64)
        ohs = []
        for h in range(HEADS):
            hs = slice(h * DH, (h + 1) * DH)
            lg = jax.lax.dot_general(qw[hs], kw[hs], (((0,), (0,)), ((), ())),
                                     preferred_element_type=jnp.float32)
            at = jax.nn.softmax(lg * SCALE, axis=-1)
            ohs.append(jax.lax.dot_general(vw[hs], at,
                                           (((1,), (1,)), ((), ())),
                                           preferred_element_type=jnp.float32))
        ow = jnp.concatenate(ohs, axis=0)
        m = mstrip[:, wloc:wloc + 1]
        osel = m * ow + (1.0 - m) * vw
        parts.append(osel.reshape(C, 8, 8))
    ao = jnp.concatenate(parts, axis=2)
    y = xt + _mm3(wo_ref[:], ao) + bo_ref[:][:, :, None]
    xn2 = _ln_cf3(y, ln2w_ref, ln2b_ref)
    t = _mm3(w1t_ref[:], xn2) + b1_ref[:][:, :, None]
    tg = jax.nn.gelu(t[:2 * C]) * t[2 * C:]
    out_ref[:, 0, :, :] = y + _mm3(w2t_ref[:], tg) + b2_ref[:][:, :, None]


def _full(shape):
    return pl.BlockSpec(shape, lambda *_: tuple(0 for _ in shape))


@jax.jit
def kernel(x, ln1_w, ln1_b, wv, bv, pred_in_w, pred_in_b, pred_sa_w,
           pred_sa_b, pred_m1, pred_b1, pred_m2, pred_b2, wq, wk, wo, bo,
           ln2_w, ln2_b, ffn_w1, ffn_b1, ffn_w2, ffn_b2):
    x4 = x.reshape(C, NWH, WS, W)
    lnw = ln1_w.reshape(C, 1)
    lnb = ln1_b.reshape(C, 1)
    piw = pred_in_w
    pib = pred_in_b.reshape(-1, 1)
    m1r = pred_m1.reshape(8, 8, 8)
    b1r = pred_b1.reshape(8, 1)
    m2d = (pred_m2[:, 0] - pred_m2[:, 1]).reshape(1, 8)

    d = pl.pallas_call(
        _pred_kernel,
        grid=(NWH,),
        in_specs=[
            pl.BlockSpec((C, 1, WS, W), lambda r: (0, r, 0, 0)),
            _full((C, 1)), _full((C, 1)), _full((24, C)), _full((24, 1)),
            _full((8, 8, 8)), _full((8, 1)), _full((1, 8)),
        ],
        out_specs=pl.BlockSpec((1, 1, NWW), lambda r: (r, 0, 0)),
        out_shape=jax.ShapeDtypeStruct((NWH, 1, NWW), jnp.float32),
    )(x4, lnw, lnb, piw, pib, m1r, b1r, m2d)

    d2 = d.reshape(18, 128)
    mask = pl.pallas_call(
        _mask_kernel,
        grid=(18,),
        in_specs=[_full((18, 128))],
        out_specs=pl.BlockSpec((1, 1, 128), lambda r: (r, 0, 0)),
        out_shape=jax.ShapeDtypeStruct((18, 1, 128), jnp.float32),
    )(d2)
    mask3 = mask.reshape(NWH, NCG, WPB)

    out = pl.pallas_call(
        _block_kernel,
        grid=(NWH, NCG),
        in_specs=[
            pl.BlockSpec((C, 1, WS, WPB * 8), lambda r, c: (0, r, 0, c)),
            _full((NWH, NCG, WPB)),
            _full((C, 1)), _full((C, 1)),
            _full((C, C)), _full((C, 1)),
            _full((24, C)), _full((24, 1)),
            _full((1, 24)), _full((1, 1)),
            _full((C, C)), _full((C, C)),
            _full((C, C)), _full((C, 1)),
            _full((C, 1)), _full((C, 1)),
            _full((4 * C, C)), _full((4 * C, 1)),
            _full((C, 2 * C)), _full((C, 1)),
        ],
        out_specs=pl.BlockSpec((C, 1, WS, WPB * 8), lambda r, c: (0, r, 0, c)),
        out_shape=jax.ShapeDtypeStruct((C, NWH, WS, W), jnp.float32),
    )(x4, mask3, lnw, lnb, wv, bv.reshape(C, 1), piw, pib, pred_sa_w,
      pred_sa_b.reshape(1, 1), wq.T, wk.T, wo, bo.reshape(C, 1),
      ln2_w.reshape(C, 1), ln2_b.reshape(C, 1), ffn_w1.T,
      ffn_b1.reshape(4 * C, 1), ffn_w2.T, ffn_b2.reshape(C, 1))

    return out.reshape(1, C, H, W)


# speedup vs baseline: 3.1515x; 3.1515x over previous
import math

import jax
import jax.numpy as jnp
from jax.experimental import pallas as pl

C = 96
H = W = 384
WS = 8
HEADS = 4
DH = C // HEADS
NWH = H // WS
NWW = W // WS
NW = NWH * NWW
K = NW // 2
EPS = 1e-6
SCALE = 1.0 / math.sqrt(float(DH))
WPB = 16
NCG = NWW // WPB


def _ln_cf3(xt, w_ref, b_ref):
    u = jnp.mean(xt, axis=0, keepdims=True)
    s = jnp.mean((xt - u) * (xt - u), axis=0, keepdims=True)
    xn = (xt - u) * jax.lax.rsqrt(s + EPS)
    return w_ref[:][:, :, None] * xn + b_ref[:][:, :, None]


def _mm3(w, x3):
    return jax.lax.dot_general(w, x3, (((1,), (0,)), ((), ())),
                               preferred_element_type=jnp.float32)


def _pred_kernel(x_ref, lnw_ref, lnb_ref, piw_ref, pib_ref, m1_ref, b1_ref,
                 m2d_ref, d_ref):
    xt = x_ref[:, 0, :, :]
    xn = _ln_cf3(xt, lnw_ref, lnb_ref)
    f = _mm3(piw_ref[:], xn) + pib_ref[:][:, :, None]
    f = jax.nn.leaky_relu(f, 0.1)
    fm = jnp.mean(f, axis=0)
    fm3 = fm.reshape(8, NWW, 8)
    h1 = jnp.zeros((8, NWW), jnp.float32) + b1_ref[:]
    for i in range(8):
        h1 = h1 + jax.lax.dot_general(m1_ref[i], fm3[i],
                                      (((0,), (1,)), ((), ())),
                                      preferred_element_type=jnp.float32)
    h1 = jax.nn.leaky_relu(h1, 0.1)
    d = jax.lax.dot_general(m2d_ref[:], h1, (((1,), (0,)), ((), ())),
                            preferred_element_type=jnp.float32)
    d_ref[0, :, :] = d


def _mask_kernel(d_ref, mask_ref):
    r = pl.program_id(0)
    dall = d_ref[:]
    di = d_ref[pl.ds(r, 1), :]
    dj3 = dall[:, :, None]
    di3 = di[None, :, :]
    jidx = (jax.lax.broadcasted_iota(jnp.int32, (18, 128, 128), 0) * 128
            + jax.lax.broadcasted_iota(jnp.int32, (18, 128, 128), 1))
    iidx = (jax.lax.broadcasted_iota(jnp.int32, (18, 128, 128), 2)
            + r * 128)
    ahead = (dj3 > di3) | ((dj3 == di3) & (jidx < iidx))
    cnt = jnp.sum(ahead.astype(jnp.float32), axis=(0, 1))
    mask_ref[0, 0, :] = (cnt < float(K)).astype(jnp.float32)


def _mm(a, b):
    return jax.lax.dot_general(a, b, (((1,), (0,)), ((), ())),
                               preferred_element_type=jnp.float32)


def _ln_pm(xp, w_ref, b_ref):
    u = jnp.mean(xp, axis=1, keepdims=True)
    s = jnp.mean((xp - u) * (xp - u), axis=1, keepdims=True)
    return (xp - u) * jax.lax.rsqrt(s + EPS) * w_ref[:] + b_ref[:]


def _block_kernel(x_ref, mask_ref, lnw_ref, lnb_ref, wvt_ref, bv_ref,
                  piwt_ref, pib_ref, pswt_ref, psb_ref, wq_ref, wk_ref,
                  wot_ref, bo_ref, ln2w_ref, ln2b_ref, w1a_ref, b1a_ref,
                  w1g_ref, b1g_ref, w2_ref, b2_ref, out_ref):
    r = pl.program_id(0)
    cg = pl.program_id(1)
    npix = WPB * 64
    parts = []
    for i in range(WS):
        ti = jax.lax.transpose(x_ref[:, 0, i, :], (1, 0))
        parts.append(ti.reshape(WPB, 1, 8, C))
    xp = jnp.concatenate(parts, axis=1).reshape(npix, C)
    xn = _ln_pm(xp, lnw_ref, lnb_ref)
    xnb = xn.astype(jnp.bfloat16)
    f = jax.nn.leaky_relu(_mm(xnb, piwt_ref[:]) + pib_ref[:], 0.1)
    sa = jax.nn.sigmoid(_mm(f.astype(jnp.bfloat16), pswt_ref[:])
                        + psb_ref[0, 0])
    mstrip = mask_ref[r, pl.ds(cg, 1), :]
    yacc = bo_ref[:] * 1.0
    for h in range(HEADS):
        hs = slice(h * DH, (h + 1) * DH)
        qh = _mm(xnb, wq_ref[:, hs])
        kh = _mm(xnb, wk_ref[:, hs])
        vh = (_mm(xnb, wvt_ref[:, hs]) + bv_ref[:, hs]) * sa
        qhb = (qh * SCALE).astype(jnp.bfloat16)
        khb = kh.astype(jnp.bfloat16)
        vhb = vh.astype(jnp.bfloat16)
        lgs = []
        for wloc in range(WPB):
            ws_ = slice(wloc * 64, wloc * 64 + 64)
            lgs.append(jax.lax.dot_general(qhb[ws_], khb[ws_],
                                           (((1,), (1,)), ((), ())),
                                           preferred_element_type=jnp.float32))
        lg = jnp.concatenate(lgs, axis=0)
        at = jax.nn.softmax(lg, axis=-1).astype(jnp.bfloat16)
        osel = []
        for wloc in range(WPB):
            ws_ = slice(wloc * 64, wloc * 64 + 64)
            oh = _mm(at[ws_], vhb[ws_])
            m = mstrip[:, wloc:wloc + 1]
            osel.append(m * oh + (1.0 - m) * vh[ws_])
        aoh = jnp.concatenate(osel, axis=0)
        yacc = yacc + _mm(aoh.astype(jnp.bfloat16), wot_ref[hs, :])
    y = xp + yacc
    xn2 = _ln_pm(y, ln2w_ref, ln2b_ref)
    xn2b = xn2.astype(jnp.bfloat16)
    a = _mm(xn2b, w1a_ref[:]) + b1a_ref[:]
    g = _mm(xn2b, w1g_ref[:]) + b1g_ref[:]
    tg = (jax.nn.gelu(a) * g).astype(jnp.bfloat16)
    o = y + _mm(tg, w2_ref[:]) + b2_ref[:]
    o4 = o.reshape(WPB, WS, 8, C)
    for i in range(WS):
        oi = o4[:, i, :, :].reshape(WPB * 8, C)
        out_ref[:, 0, i, :] = jax.lax.transpose(oi, (1, 0))


def _full(shape):
    return pl.BlockSpec(shape, lambda *_: tuple(0 for _ in shape))


@jax.jit
def kernel(x, ln1_w, ln1_b, wv, bv, pred_in_w, pred_in_b, pred_sa_w,
           pred_sa_b, pred_m1, pred_b1, pred_m2, pred_b2, wq, wk, wo, bo,
           ln2_w, ln2_b, ffn_w1, ffn_b1, ffn_w2, ffn_b2):
    x4 = x.reshape(C, NWH, WS, W)
    lnw = ln1_w.reshape(C, 1)
    lnb = ln1_b.reshape(C, 1)
    piw = pred_in_w
    pib = pred_in_b.reshape(-1, 1)
    m1r = pred_m1.reshape(8, 8, 8)
    b1r = pred_b1.reshape(8, 1)
    m2d = (pred_m2[:, 0] - pred_m2[:, 1]).reshape(1, 8)

    d = pl.pallas_call(
        _pred_kernel,
        grid=(NWH,),
        in_specs=[
            pl.BlockSpec((C, 1, WS, W), lambda r: (0, r, 0, 0)),
            _full((C, 1)), _full((C, 1)), _full((24, C)), _full((24, 1)),
            _full((8, 8, 8)), _full((8, 1)), _full((1, 8)),
        ],
        out_specs=pl.BlockSpec((1, 1, NWW), lambda r: (r, 0, 0)),
        out_shape=jax.ShapeDtypeStruct((NWH, 1, NWW), jnp.float32),
    )(x4, lnw, lnb, piw, pib, m1r, b1r, m2d)

    d2 = d.reshape(18, 128)
    mask = pl.pallas_call(
        _mask_kernel,
        grid=(18,),
        in_specs=[_full((18, 128))],
        out_specs=pl.BlockSpec((1, 1, 128), lambda r: (r, 0, 0)),
        out_shape=jax.ShapeDtypeStruct((18, 1, 128), jnp.float32),
    )(d2)
    mask3 = mask.reshape(NWH, NCG, WPB)

    out = pl.pallas_call(
        _block_kernel,
        grid=(NWH, NCG),
        in_specs=[
            pl.BlockSpec((C, 1, WS, WPB * 8), lambda r, c: (0, r, 0, c)),
            _full((NWH, NCG, WPB)),
            _full((1, C)), _full((1, C)),
            _full((C, C)), _full((1, C)),
            _full((C, 24)), _full((1, 24)),
            _full((24, 1)), _full((1, 1)),
            _full((C, C)), _full((C, C)),
            _full((C, C)), _full((1, C)),
            _full((1, C)), _full((1, C)),
            _full((C, 2 * C)), _full((1, 2 * C)),
            _full((C, 2 * C)), _full((1, 2 * C)),
            _full((2 * C, C)), _full((1, C)),
        ],
        out_specs=pl.BlockSpec((C, 1, WS, WPB * 8), lambda r, c: (0, r, 0, c)),
        out_shape=jax.ShapeDtypeStruct((C, NWH, WS, W), jnp.float32),
    )(x4, mask3, ln1_w.reshape(1, C), ln1_b.reshape(1, C),
      wv.T.astype(jnp.bfloat16), bv.reshape(1, C),
      pred_in_w.T.astype(jnp.bfloat16), pred_in_b.reshape(1, 24),
      pred_sa_w.T.astype(jnp.bfloat16), pred_sa_b.reshape(1, 1),
      wq.astype(jnp.bfloat16), wk.astype(jnp.bfloat16),
      wo.T.astype(jnp.bfloat16), bo.reshape(1, C),
      ln2_w.reshape(1, C), ln2_b.reshape(1, C),
      ffn_w1[:, :2 * C].astype(jnp.bfloat16),
      ffn_b1[:2 * C].reshape(1, 2 * C),
      ffn_w1[:, 2 * C:].astype(jnp.bfloat16),
      ffn_b1[2 * C:].reshape(1, 2 * C),
      ffn_w2.astype(jnp.bfloat16), ffn_b2.reshape(1, C))

    return out.reshape(1, C, H, W)
